# baseline (speedup 1.0000x reference)
import functools

import jax
import jax.numpy as jnp
from jax import lax
from jax.experimental import pallas as pl
from jax.experimental.pallas import tpu as pltpu
from jax.experimental.pallas import tpu_sc as plsc

_RANK = 32
_B = 16384
_LAMBD = 0.01
_NW = 32
_BPW = _B // _NW
_CHUNK = 128
_NCHUNK = _BPW // _CHUNK


def _sc_body(idx0_hbm, idx1_hbm, idx2_hbm, y_hbm, u0_hbm, u1_hbm, u2_hbm,
             out_hbm, idx_v, rows0_v, rows1_v, rows2_v, y_v, part_v, sem):
    wid = lax.axis_index("s") * 2 + lax.axis_index("c")
    pltpu.sync_copy(idx0_hbm.at[pl.ds(wid * _NCHUNK, _NCHUNK)], idx_v.at[0])
    pltpu.sync_copy(idx1_hbm.at[pl.ds(wid * _NCHUNK, _NCHUNK)], idx_v.at[1])
    pltpu.sync_copy(idx2_hbm.at[pl.ds(wid * _NCHUNK, _NCHUNK)], idx_v.at[2])
    pltpu.sync_copy(y_hbm.at[wid], y_v)

    copies = []
    for m, (u_hbm, rows_v) in enumerate(
        ((u0_hbm, rows0_v), (u1_hbm, rows1_v), (u2_hbm, rows2_v))):
        for j in range(_NCHUNK):
            copies.append(pltpu.async_copy(
                u_hbm.at[idx_v.at[m, j]],
                rows_v.at[pl.ds(j * _CHUNK, _CHUNK)], sem))
    for c in copies:
        c.wait()

    lane = lax.iota(jnp.int32, 16)
    zero = jnp.zeros((16,), jnp.float32)

    def group(g, carry):
        regacc, lacc = carry
        ridx = g * 16 + lane
        pred = zero
        sq = zero
        for k in range(_RANK):
            cidx = jnp.bitwise_and(lane + k, _RANK - 1)
            a = plsc.load_gather(rows0_v, [ridx, cidx])
            b = plsc.load_gather(rows1_v, [ridx, cidx])
            c = plsc.load_gather(rows2_v, [ridx, cidx])
            pred = pred + a * b * c
            sq = sq + (a * a + b * b + c * c)
        e = pred - y_v[pl.ds(g * 16, 16)]
        return regacc + sq, lacc + e * e

    regacc, lacc = lax.fori_loop(0, _BPW // 16, group, (zero, zero))
    part_v[...] = lacc + _LAMBD * regacc
    pltpu.sync_copy(part_v, out_hbm.at[wid])


@jax.jit
def kernel(idx0, idx1, idx2, y, U0, U1, U2):
    mesh = plsc.VectorSubcoreMesh(core_axis_name="c", subcore_axis_name="s",
                                  num_cores=2, num_subcores=16)
    sc = functools.partial(
        pl.kernel, mesh=mesh,
        compiler_params=pltpu.CompilerParams(
            needs_layout_passes=False, use_tc_tiling_on_sc=False),
        out_type=jax.ShapeDtypeStruct((_NW, 16), jnp.float32),
        scratch_types=[
            pltpu.VMEM((3, _NCHUNK, _CHUNK), jnp.int32),
            pltpu.VMEM((_BPW, _RANK), jnp.float32),
            pltpu.VMEM((_BPW, _RANK), jnp.float32),
            pltpu.VMEM((_BPW, _RANK), jnp.float32),
            pltpu.VMEM((_BPW,), jnp.float32),
            pltpu.VMEM((16,), jnp.float32),
            pltpu.SemaphoreType.DMA,
        ],
    )(_sc_body)
    parts = sc(idx0.reshape(_NW * _NCHUNK, _CHUNK),
               idx1.reshape(_NW * _NCHUNK, _CHUNK),
               idx2.reshape(_NW * _NCHUNK, _CHUNK),
               y.reshape(_NW, _BPW), U0, U1, U2)
    return jnp.sum(parts)


# speedup vs baseline: 1.2815x; 1.2815x over previous
import functools

import jax
import jax.numpy as jnp
from jax import lax
from jax.experimental import pallas as pl
from jax.experimental.pallas import tpu as pltpu
from jax.experimental.pallas import tpu_sc as plsc

_RANK = 32
_B = 16384
_LAMBD = 0.01
_NW = 32
_BPW = _B // _NW
_P = 32


def _sc_body(idx0_hbm, idx1_hbm, idx2_hbm, y_hbm, u0_hbm, u1_hbm, u2_hbm,
             out_hbm, idx_v, y_v, s0_v, s1_v, s2_v, part_v, sem):
    wid = lax.axis_index("s") * 2 + lax.axis_index("c")
    base = wid * _BPW
    for j in range(_BPW // 128):
        pltpu.sync_copy(idx0_hbm.at[pl.ds(base + j * 128, 128)],
                        idx_v.at[0, j, 0])
        pltpu.sync_copy(idx1_hbm.at[pl.ds(base + j * 128, 128)],
                        idx_v.at[1, j, 0])
        pltpu.sync_copy(idx2_hbm.at[pl.ds(base + j * 128, 128)],
                        idx_v.at[2, j, 0])
    pltpu.sync_copy(y_hbm.at[pl.ds(base, _BPW)], y_v)

    lane = lax.iota(jnp.int32, 16)
    zero = jnp.zeros((16,), jnp.float32)
    seven = jnp.full((16,), 7, jnp.int32)

    def phase(p, carry):
        regacc, lacc = carry
        copies = []
        ivecs = []
        for h in range(2):
            ch = p * 2 + h
            jj = ch // 8
            off = (ch % 8) * 16
            i0 = idx_v[0, jj, 0, pl.ds(off, 16)]
            i1 = idx_v[1, jj, 0, pl.ds(off, 16)]
            i2 = idx_v[2, jj, 0, pl.ds(off, 16)]
            g0 = lax.shift_right_logical(i0, 3) * 8
            g1 = lax.shift_right_logical(i1, 3) * 8
            g2 = lax.shift_right_logical(i2, 3) * 8
            for j in range(16):
                for u_hbm, s_v, g in ((u0_hbm, s0_v, g0), (u1_hbm, s1_v, g1),
                                      (u2_hbm, s2_v, g2)):
                    g8 = pl.multiple_of(g[j], 8)
                    copies.append(pltpu.async_copy(
                        u_hbm.at[pl.ds(g8, 8), :], s_v.at[h * 16 + j], sem))
            ivecs.append((i0, i1, i2))
        for c in copies:
            c.wait()

        for h in range(2):
            i0, i1, i2 = ivecs[h]
            item = h * 16 + lane
            s0 = jnp.bitwise_and(i0, seven)
            s1 = jnp.bitwise_and(i1, seven)
            s2 = jnp.bitwise_and(i2, seven)
            pred = zero
            sq = zero
            for k in range(_RANK):
                kk = jnp.bitwise_and(lane + k, _RANK - 1)
                a = plsc.load_gather(s0_v, [item, s0, kk])
                b = plsc.load_gather(s1_v, [item, s1, kk])
                cc = plsc.load_gather(s2_v, [item, s2, kk])
                pred = pred + a * b * cc
                sq = sq + (a * a + b * b + cc * cc)
            e = pred - y_v[pl.ds(p * _P + h * 16, 16)]
            regacc = regacc + sq
            lacc = lacc + e * e
        return regacc, lacc

    regacc, lacc = lax.fori_loop(0, _BPW // _P, phase, (zero, zero))
    part = lacc + _LAMBD * regacc
    plsc.store_scatter(part_v, [jnp.zeros((16,), jnp.int32), lane], part)
    pltpu.sync_copy(part_v.at[pl.ds(0, 8), :],
                    out_hbm.at[pl.ds(wid * 8, 8), :])


@jax.jit
def kernel(idx0, idx1, idx2, y, U0, U1, U2):
    mesh = plsc.VectorSubcoreMesh(core_axis_name="c", subcore_axis_name="s",
                                  num_cores=2, num_subcores=16)
    sc = functools.partial(
        pl.kernel, mesh=mesh,
        compiler_params=pltpu.CompilerParams(
            needs_layout_passes=False, use_tc_tiling_on_sc=True),
        out_type=jax.ShapeDtypeStruct((_NW * 8, 128), jnp.float32),
        scratch_types=[
            pltpu.VMEM((3, _BPW // 128, 1, 128), jnp.int32),
            pltpu.VMEM((_BPW,), jnp.float32),
            pltpu.VMEM((_P, 8, _RANK), jnp.float32),
            pltpu.VMEM((_P, 8, _RANK), jnp.float32),
            pltpu.VMEM((_P, 8, _RANK), jnp.float32),
            pltpu.VMEM((8, 128), jnp.float32),
            pltpu.SemaphoreType.DMA,
        ],
    )(_sc_body)
    parts = sc(idx0, idx1, idx2, y, U0, U1, U2)
    return jnp.sum(parts.reshape(_NW, 8, 128)[:, 0, :16])
